# baseline (speedup 1.0000x reference)
import functools

import jax
import jax.numpy as jnp
from jax import lax
from jax.experimental import pallas as pl
from jax.experimental.pallas import tpu as pltpu
from jax.experimental.pallas import tpu_sc as plsc

N_NODES = 10000
N_EDGES = 160000
EMB = 256
HALF = 128
N_LAYERS = 3

NC = 2
NS = 16
CHUNK = 128

EP = 163840
E_PER_TILE = EP // NS
ECHUNK = 80
E_CHUNKS = E_PER_TILE // ECHUNK

AE = N_NODES * 9
AEP = 98304
A_PER_TILE = AEP // NS
A_CHUNKS = A_PER_TILE // CHUNK

NPAD = 10400
ZSTRIPE = 648
OUT_STRIPE = 624

ATAB_ROWS = 960
COMBO_ROWS = 128


def _vec_mesh():
    return plsc.VectorSubcoreMesh(core_axis_name="c", subcore_axis_name="s")


def _zero_rows(rows, n):
    z = jnp.zeros((16,), jnp.float32)

    @pl.loop(0, n)
    def _(i):
        for j in range(8):
            rows[i, pl.ds(j * 16, 16)] = z


def _zero_stripe(rows, n, agg_sh, t):
    base = t * ZSTRIPE

    @pl.loop(0, ZSTRIPE // n)
    def _(i):
        pltpu.sync_copy(rows.at[pl.ds(0, n)],
                        agg_sh.at[pl.ds(base + i * n, n)])

    rem = ZSTRIPE % n
    if rem:
        pltpu.sync_copy(rows.at[pl.ds(0, rem)],
                        agg_sh.at[pl.ds(base + ZSTRIPE - rem, rem)])

    @pl.when(t == NS - 1)
    def _():
        pltpu.sync_copy(rows.at[pl.ds(0, 32)],
                        agg_sh.at[pl.ds(NS * ZSTRIPE, 32)])


def _copy_out(agg_sh, out_hbm, s, t, half_rows):
    ob = s * half_rows + t * OUT_STRIPE
    lb = t * OUT_STRIPE

    @pl.loop(0, 4)
    def _(i):
        pltpu.sync_copy(agg_sh.at[pl.ds(lb + i * CHUNK, CHUNK)],
                        out_hbm.at[pl.ds(ob + i * CHUNK, CHUNK)])

    pltpu.sync_copy(agg_sh.at[pl.ds(lb + 512, 112)],
                    out_hbm.at[pl.ds(ob + 512, 112)])

    @pl.when(t == NS - 1)
    def _():
        pltpu.sync_copy(agg_sh.at[pl.ds(9984, 16)],
                        out_hbm.at[pl.ds(s * half_rows + 9984, 16)])


def _atom_encode(atabf, asrc, adst):
    @functools.partial(
        pl.kernel,
        out_type=jax.ShapeDtypeStruct((2 * N_NODES, HALF), jnp.float32),
        mesh=_vec_mesh(),
        scratch_types=[
            pltpu.VMEM((CHUNK,), jnp.int32),
            pltpu.VMEM((CHUNK,), jnp.int32),
            pltpu.VMEM((CHUNK,), jnp.int32),
            pltpu.VMEM((CHUNK, HALF), jnp.float32),
            pltpu.VMEM_SHARED((NPAD, HALF), jnp.float32),
        ],
    )
    def k(atab_hbm, asrc_hbm, adst_hbm, h_hbm, aidx, aidx2, didx, rows, agg_sh):
        s = lax.axis_index("c")
        t = lax.axis_index("s")
        _zero_rows(rows, CHUNK)
        _zero_stripe(rows, CHUNK, agg_sh, t)
        plsc.subcore_barrier()

        eb = t * A_PER_TILE
        soff = s * ATAB_ROWS

        @pl.loop(0, A_CHUNKS)
        def _(kk):
            cb = eb + kk * CHUNK
            pltpu.sync_copy(asrc_hbm.at[pl.ds(cb, CHUNK)], aidx)
            pltpu.sync_copy(adst_hbm.at[pl.ds(cb, CHUNK)], didx)

            @pl.loop(0, CHUNK // 16)
            def _(g):
                sl = pl.ds(g * 16, 16)
                aidx2[sl] = aidx[sl] + soff

            pltpu.sync_copy(atab_hbm.at[aidx2], rows)
            pltpu.sync_copy(rows, agg_sh.at[didx], add=True)

        plsc.subcore_barrier()
        _copy_out(agg_sh, h_hbm, s, t, N_NODES)

    return k(atabf, asrc, adst)


def _edge_layer(h_flat, esrc, edst, ecode, combo_rep):
    ib = lambda: pltpu.VMEM((ECHUNK,), jnp.int32)
    rb = lambda: pltpu.VMEM((ECHUNK, HALF), jnp.float32)

    @functools.partial(
        pl.kernel,
        out_type=jax.ShapeDtypeStruct((2 * NPAD, HALF), jnp.float32),
        mesh=_vec_mesh(),
        scratch_types=[
            ib(), ib(), ib(), ib(), ib(), ib(),
            rb(), rb(), rb(), rb(),
            pltpu.VMEM_SHARED((NPAD, HALF), jnp.float32),
            pltpu.SemaphoreType.DMA,
            pltpu.SemaphoreType.DMA,
            pltpu.SemaphoreType.DMA,
            pltpu.SemaphoreType.DMA,
            pltpu.SemaphoreType.DMA,
            pltpu.SemaphoreType.DMA,
            pltpu.SemaphoreType.DMA,
            pltpu.SemaphoreType.DMA,
        ],
    )
    def k(h_hbm, esrc_hbm, edst_hbm, ecode_hbm, combo_hbm, agg_hbm,
          sA, dA, cA, sB, dB, cB, rowsA, crowsA, rowsB, crowsB,
          agg_sh, semIA, semIB, semDA, semDB, semGA, semGB, semSA, semSB):
        s = lax.axis_index("c")
        t = lax.axis_index("s")
        _zero_rows(rowsA, ECHUNK)
        _zero_stripe(rowsA, ECHUNK, agg_sh, t)
        plsc.subcore_barrier()

        eb = t * E_PER_TILE
        hoff = s * N_NODES
        coff = (s * NS + t) * COMBO_ROWS

        def issue_idx_sc(c, sx, cx, semI):
            cbase = eb + c * ECHUNK
            pltpu.async_copy(esrc_hbm.at[pl.ds(cbase, ECHUNK)], sx, semI)
            pltpu.async_copy(ecode_hbm.at[pl.ds(cbase, ECHUNK)], cx, semI)

        def wait_idx_sc(sx, cx, semI):
            pltpu.make_async_copy(esrc_hbm.at[pl.ds(0, ECHUNK)], sx,
                                  semI).wait()
            pltpu.make_async_copy(ecode_hbm.at[pl.ds(0, ECHUNK)], cx,
                                  semI).wait()

        def issue_idx_d(c, dx, semD):
            pltpu.async_copy(edst_hbm.at[pl.ds(eb + c * ECHUNK, ECHUNK)],
                             dx, semD)

        def wait_idx_d(dx, semD):
            pltpu.make_async_copy(edst_hbm.at[pl.ds(0, ECHUNK)], dx,
                                  semD).wait()

        def add_offsets(sx, cx):
            @pl.loop(0, ECHUNK // 16)
            def _(g):
                sl = pl.ds(g * 16, 16)
                sx[sl] = sx[sl] + hoff
                cx[sl] = cx[sl] + coff

        def issue_gathers(sx, cx, rows, crows, semG):
            pltpu.async_copy(h_hbm.at[sx], rows, semG)
            pltpu.async_copy(combo_hbm.at[cx], crows, semG)

        def wait_gathers(sx, cx, rows, crows, semG):
            pltpu.make_async_copy(h_hbm.at[sx], rows, semG).wait()
            pltpu.make_async_copy(combo_hbm.at[cx], crows, semG).wait()

        def drain_scatter(rows, dx, semS):
            pltpu.make_async_copy(rows, agg_sh.at[dx], semS).wait()

        def step(c, X, Y, not_first, not_last2, y_has_next):
            sx, dx, cx, rows, crows, semI, semD, semG, semS = X
            sy, dy, cy, rows_y, crows_y, semIY, semDY, semGY, semSY = Y

            @pl.when(y_has_next)
            def _():
                wait_idx_sc(sy, cy, semIY)
                add_offsets(sy, cy)

            @pl.when(not_first)
            def _():
                drain_scatter(rows_y, dy, semSY)

            @pl.when(y_has_next)
            def _():
                issue_idx_d(c + 1, dy, semDY)
                issue_gathers(sy, cy, rows_y, crows_y, semGY)

            wait_gathers(sx, cx, rows, crows, semG)

            @pl.when(not_last2)
            def _():
                issue_idx_sc(c + 2, sx, cx, semI)

            @plsc.parallel_loop(0, ECHUNK, unroll=4)
            def _(i):
                for q in range(8):
                    sl = pl.ds(q * 16, 16)
                    rows[i, sl] = rows[i, sl] * crows[i, sl]

            wait_idx_d(dx, semD)
            pltpu.async_copy(rows, agg_sh.at[dx], semS, add=True)

        A = (sA, dA, cA, rowsA, crowsA, semIA, semDA, semGA, semSA)
        B = (sB, dB, cB, rowsB, crowsB, semIB, semDB, semGB, semSB)

        issue_idx_sc(0, sA, cA, semIA)
        issue_idx_d(0, dA, semDA)
        wait_idx_sc(sA, cA, semIA)
        add_offsets(sA, cA)
        issue_gathers(sA, cA, rowsA, crowsA, semGA)
        issue_idx_sc(1, sB, cB, semIB)

        TRUE = jnp.bool_(True)

        @pl.loop(0, E_CHUNKS // 2)
        def _(kk):
            nl = kk < E_CHUNKS // 2 - 1
            step(2 * kk, A, B, not_first=kk > 0, not_last2=nl,
                 y_has_next=TRUE)
            step(2 * kk + 1, B, A, not_first=TRUE, not_last2=nl,
                 y_has_next=nl)

        drain_scatter(rowsB, dB, semSB)

        plsc.subcore_barrier()
        _copy_out(agg_sh, agg_hbm, s, t, NPAD)

    return k(h_flat, esrc, edst, ecode, combo_rep)


def _tc_linear(agg, W, b):
    def body(a0_ref, a1_ref, w_ref, b_ref, o_ref):
        x0 = a0_ref[...]
        x1 = a1_ref[...]
        y = jnp.dot(x0, w_ref[0:HALF, :], preferred_element_type=jnp.float32)
        y = y + jnp.dot(x1, w_ref[HALF:EMB, :],
                        preferred_element_type=jnp.float32)
        y = jnp.maximum(y + b_ref[...], 0.0)
        o_ref[0, :, :] = y[:, 0:HALF]
        o_ref[1, :, :] = y[:, HALF:EMB]

    return pl.pallas_call(
        body,
        grid=(25,),
        in_specs=[
            pl.BlockSpec((400, HALF), lambda i: (i, 0)),
            pl.BlockSpec((400, HALF), lambda i: (i + NPAD // 400, 0)),
            pl.BlockSpec((EMB, EMB), lambda i: (0, 0)),
            pl.BlockSpec((1, EMB), lambda i: (0, 0)),
        ],
        out_specs=pl.BlockSpec((2, 400, HALF), lambda i: (0, i, 0)),
        out_shape=jax.ShapeDtypeStruct((2, N_NODES, HALF), jnp.float32),
    )(agg, agg, W, b.reshape(1, EMB))


def _tc_last(agg, W, b, clf_W):
    def body(a0_ref, a1_ref, w_ref, b_ref, cw_ref, o_ref):
        i = pl.program_id(0)
        x0 = a0_ref[...]
        x1 = a1_ref[...]
        y = jnp.dot(x0, w_ref[0:HALF, :], preferred_element_type=jnp.float32)
        y = y + jnp.dot(x1, w_ref[HALF:EMB, :],
                        preferred_element_type=jnp.float32)
        y = jnp.maximum(y + b_ref[...], 0.0)
        p = y * cw_ref[...]
        acc = p[0:8, :]
        for r in range(1, 50):
            acc = acc + p[8 * r:8 * (r + 1), :]

        @pl.when(i == 0)
        def _():
            o_ref[...] = jnp.zeros((8, EMB), jnp.float32)

        o_ref[...] += acc

    return pl.pallas_call(
        body,
        grid=(25,),
        in_specs=[
            pl.BlockSpec((400, HALF), lambda i: (i, 0)),
            pl.BlockSpec((400, HALF), lambda i: (i + NPAD // 400, 0)),
            pl.BlockSpec((EMB, EMB), lambda i: (0, 0)),
            pl.BlockSpec((1, EMB), lambda i: (0, 0)),
            pl.BlockSpec((1, EMB), lambda i: (0, 0)),
        ],
        out_specs=pl.BlockSpec((8, EMB), lambda i: (0, 0)),
        out_shape=jax.ShapeDtypeStruct((8, EMB), jnp.float32),
    )(agg, agg, W, b.reshape(1, EMB), clf_W.reshape(1, EMB))


def kernel(node_feat, edge_feat, edge_index, atom_tables, bond_tables,
           lin_W, lin_b, clf_W, clf_b):
    i32 = jnp.int32

    asrc = (node_feat.astype(i32)
            + 100 * jnp.arange(9, dtype=i32)[None, :]).reshape(-1)
    adst = jnp.repeat(jnp.arange(N_NODES, dtype=i32), 9,
                      total_repeat_length=AE)
    apad = jnp.arange(AEP - AE, dtype=i32)
    asrc = jnp.concatenate([asrc, 900 + apad % 60])
    adst = jnp.concatenate([adst, N_NODES + apad % 16])

    atab = atom_tables.reshape(900, 2, HALF).transpose(1, 0, 2)
    atabf = jnp.concatenate(
        [atab, jnp.zeros((2, ATAB_ROWS - 900, HALF), jnp.float32)],
        axis=1).reshape(2 * ATAB_ROWS, HALF)

    ef = edge_feat.astype(i32)
    ecode = ef[:, 0] * 25 + ef[:, 1] * 5 + ef[:, 2]
    epad = jnp.arange(EP - N_EDGES, dtype=i32)
    esrc = jnp.concatenate([edge_index[0].astype(i32), epad % N_NODES])
    edst = jnp.concatenate([edge_index[1].astype(i32), N_NODES + epad % 16])
    ecode = jnp.concatenate([ecode, jnp.full((EP - N_EDGES,), 125, i32)])

    cb = (bond_tables[:, 0][:, :, None, None, :]
          + bond_tables[:, 1][:, None, :, None, :]
          + bond_tables[:, 2][:, None, None, :, :]).reshape(3, 125, EMB)
    cb = jnp.concatenate(
        [cb, jnp.zeros((3, COMBO_ROWS - 125, EMB), jnp.float32)], axis=1)
    cb = cb.reshape(3, COMBO_ROWS, 2, HALF).transpose(0, 2, 1, 3)
    cb = jnp.broadcast_to(cb[:, :, None], (3, 2, NS, COMBO_ROWS, HALF))
    combo_rep = cb.reshape(3, 2 * NS * COMBO_ROWS, HALF)

    h_flat = _atom_encode(atabf, asrc, adst)
    out8 = None
    for l in range(N_LAYERS):
        agg = _edge_layer(h_flat, esrc, edst, ecode, combo_rep[l])
        if l < N_LAYERS - 1:
            h3d = _tc_linear(agg, lin_W[l], lin_b[l])
            h_flat = h3d.reshape(2 * N_NODES, HALF)
        else:
            out8 = _tc_last(agg, lin_W[l], lin_b[l], clf_W)

    return jnp.sum(out8).reshape(1) + clf_b


# speedup vs baseline: 1.0018x; 1.0018x over previous
import functools

import jax
import jax.numpy as jnp
from jax import lax
from jax.experimental import pallas as pl
from jax.experimental.pallas import tpu as pltpu
from jax.experimental.pallas import tpu_sc as plsc

N_NODES = 10000
N_EDGES = 160000
EMB = 256
HALF = 128
N_LAYERS = 3

NC = 2
NS = 16
CHUNK = 128

EP = 163840
E_PER_TILE = EP // NS
ECHUNK = 64
E_CHUNKS = E_PER_TILE // ECHUNK

AE = N_NODES * 9
AEP = 98304
A_PER_TILE = AEP // NS
A_CHUNKS = A_PER_TILE // CHUNK

NPAD = 10400
ZSTRIPE = 648
OUT_STRIPE = 624

ATAB_ROWS = 960
COMBO_ROWS = 128


def _vec_mesh():
    return plsc.VectorSubcoreMesh(core_axis_name="c", subcore_axis_name="s")


def _zero_rows(rows, n):
    z = jnp.zeros((16,), jnp.float32)

    @pl.loop(0, n)
    def _(i):
        for j in range(8):
            rows[i, pl.ds(j * 16, 16)] = z


def _zero_stripe(rows, n, agg_sh, t):
    base = t * ZSTRIPE

    @pl.loop(0, ZSTRIPE // n)
    def _(i):
        pltpu.sync_copy(rows.at[pl.ds(0, n)],
                        agg_sh.at[pl.ds(base + i * n, n)])

    rem = ZSTRIPE % n
    if rem:
        pltpu.sync_copy(rows.at[pl.ds(0, rem)],
                        agg_sh.at[pl.ds(base + ZSTRIPE - rem, rem)])

    @pl.when(t == NS - 1)
    def _():
        pltpu.sync_copy(rows.at[pl.ds(0, 32)],
                        agg_sh.at[pl.ds(NS * ZSTRIPE, 32)])


def _copy_out(agg_sh, out_hbm, s, t, half_rows):
    ob = s * half_rows + t * OUT_STRIPE
    lb = t * OUT_STRIPE

    @pl.loop(0, 4)
    def _(i):
        pltpu.sync_copy(agg_sh.at[pl.ds(lb + i * CHUNK, CHUNK)],
                        out_hbm.at[pl.ds(ob + i * CHUNK, CHUNK)])

    pltpu.sync_copy(agg_sh.at[pl.ds(lb + 512, 112)],
                    out_hbm.at[pl.ds(ob + 512, 112)])

    @pl.when(t == NS - 1)
    def _():
        pltpu.sync_copy(agg_sh.at[pl.ds(9984, 16)],
                        out_hbm.at[pl.ds(s * half_rows + 9984, 16)])


def _atom_encode(atabf, asrc, adst):
    @functools.partial(
        pl.kernel,
        out_type=jax.ShapeDtypeStruct((2 * N_NODES, HALF), jnp.float32),
        mesh=_vec_mesh(),
        scratch_types=[
            pltpu.VMEM((CHUNK,), jnp.int32),
            pltpu.VMEM((CHUNK,), jnp.int32),
            pltpu.VMEM((CHUNK,), jnp.int32),
            pltpu.VMEM((CHUNK, HALF), jnp.float32),
            pltpu.VMEM_SHARED((NPAD, HALF), jnp.float32),
        ],
    )
    def k(atab_hbm, asrc_hbm, adst_hbm, h_hbm, aidx, aidx2, didx, rows, agg_sh):
        s = lax.axis_index("c")
        t = lax.axis_index("s")
        _zero_rows(rows, CHUNK)
        _zero_stripe(rows, CHUNK, agg_sh, t)
        plsc.subcore_barrier()

        eb = t * A_PER_TILE
        soff = s * ATAB_ROWS

        @pl.loop(0, A_CHUNKS)
        def _(kk):
            cb = eb + kk * CHUNK
            pltpu.sync_copy(asrc_hbm.at[pl.ds(cb, CHUNK)], aidx)
            pltpu.sync_copy(adst_hbm.at[pl.ds(cb, CHUNK)], didx)

            @pl.loop(0, CHUNK // 16)
            def _(g):
                sl = pl.ds(g * 16, 16)
                aidx2[sl] = aidx[sl] + soff

            pltpu.sync_copy(atab_hbm.at[aidx2], rows)
            pltpu.sync_copy(rows, agg_sh.at[didx], add=True)

        plsc.subcore_barrier()
        _copy_out(agg_sh, h_hbm, s, t, N_NODES)

    return k(atabf, asrc, adst)


def _edge_layer(h_flat, esrc, edst, ecode, combo_rep):
    ib = lambda: pltpu.VMEM((ECHUNK,), jnp.int32)
    rb = lambda: pltpu.VMEM((ECHUNK, HALF), jnp.float32)

    @functools.partial(
        pl.kernel,
        out_type=jax.ShapeDtypeStruct((2 * NPAD, HALF), jnp.float32),
        mesh=_vec_mesh(),
        scratch_types=[
            ib(), ib(), ib(), ib(), ib(), ib(),
            rb(), rb(), rb(), rb(),
            pltpu.VMEM_SHARED((NPAD, HALF), jnp.float32),
            pltpu.SemaphoreType.DMA,
            pltpu.SemaphoreType.DMA,
            pltpu.SemaphoreType.DMA,
            pltpu.SemaphoreType.DMA,
            pltpu.SemaphoreType.DMA,
            pltpu.SemaphoreType.DMA,
            pltpu.SemaphoreType.DMA,
            pltpu.SemaphoreType.DMA,
        ],
    )
    def k(h_hbm, esrc_hbm, edst_hbm, ecode_hbm, combo_hbm, agg_hbm,
          sA, dA, cA, sB, dB, cB, rowsA, crowsA, rowsB, crowsB,
          agg_sh, semIA, semIB, semDA, semDB, semGA, semGB, semSA, semSB):
        s = lax.axis_index("c")
        t = lax.axis_index("s")
        _zero_rows(rowsA, ECHUNK)
        _zero_stripe(rowsA, ECHUNK, agg_sh, t)
        plsc.subcore_barrier()

        eb = t * E_PER_TILE
        hoff = s * N_NODES
        coff = (s * NS + t) * COMBO_ROWS

        def issue_idx_sc(c, sx, cx, semI):
            cbase = eb + c * ECHUNK
            pltpu.async_copy(esrc_hbm.at[pl.ds(cbase, ECHUNK)], sx, semI)
            pltpu.async_copy(ecode_hbm.at[pl.ds(cbase, ECHUNK)], cx, semI)

        def wait_idx_sc(sx, cx, semI):
            pltpu.make_async_copy(esrc_hbm.at[pl.ds(0, ECHUNK)], sx,
                                  semI).wait()
            pltpu.make_async_copy(ecode_hbm.at[pl.ds(0, ECHUNK)], cx,
                                  semI).wait()

        def issue_idx_d(c, dx, semD):
            pltpu.async_copy(edst_hbm.at[pl.ds(eb + c * ECHUNK, ECHUNK)],
                             dx, semD)

        def wait_idx_d(dx, semD):
            pltpu.make_async_copy(edst_hbm.at[pl.ds(0, ECHUNK)], dx,
                                  semD).wait()

        def add_offsets(sx, cx):
            @pl.loop(0, ECHUNK // 16)
            def _(g):
                sl = pl.ds(g * 16, 16)
                sx[sl] = sx[sl] + hoff
                cx[sl] = cx[sl] + coff

        def issue_gathers(sx, cx, rows, crows, semG):
            pltpu.async_copy(h_hbm.at[sx], rows, semG)
            pltpu.async_copy(combo_hbm.at[cx], crows, semG)

        def wait_gathers(sx, cx, rows, crows, semG):
            pltpu.make_async_copy(h_hbm.at[sx], rows, semG).wait()
            pltpu.make_async_copy(combo_hbm.at[cx], crows, semG).wait()

        def drain_scatter(rows, dx, semS):
            pltpu.make_async_copy(rows, agg_sh.at[dx], semS).wait()

        def step(c, X, Y, not_first, not_last2, y_has_next):
            sx, dx, cx, rows, crows, semI, semD, semG, semS = X
            sy, dy, cy, rows_y, crows_y, semIY, semDY, semGY, semSY = Y

            @pl.when(y_has_next)
            def _():
                wait_idx_sc(sy, cy, semIY)
                add_offsets(sy, cy)

            @pl.when(not_first)
            def _():
                drain_scatter(rows_y, dy, semSY)

            @pl.when(y_has_next)
            def _():
                issue_idx_d(c + 1, dy, semDY)
                issue_gathers(sy, cy, rows_y, crows_y, semGY)

            wait_gathers(sx, cx, rows, crows, semG)

            @pl.when(not_last2)
            def _():
                issue_idx_sc(c + 2, sx, cx, semI)

            @pl.loop(0, ECHUNK)
            def _(i):
                for q in range(8):
                    sl = pl.ds(q * 16, 16)
                    rows[i, sl] = rows[i, sl] * crows[i, sl]

            wait_idx_d(dx, semD)
            pltpu.async_copy(rows, agg_sh.at[dx], semS, add=True)

        A = (sA, dA, cA, rowsA, crowsA, semIA, semDA, semGA, semSA)
        B = (sB, dB, cB, rowsB, crowsB, semIB, semDB, semGB, semSB)

        issue_idx_sc(0, sA, cA, semIA)
        issue_idx_d(0, dA, semDA)
        wait_idx_sc(sA, cA, semIA)
        add_offsets(sA, cA)
        issue_gathers(sA, cA, rowsA, crowsA, semGA)
        issue_idx_sc(1, sB, cB, semIB)

        TRUE = jnp.bool_(True)

        @pl.loop(0, E_CHUNKS // 2)
        def _(kk):
            nl = kk < E_CHUNKS // 2 - 1
            step(2 * kk, A, B, not_first=kk > 0, not_last2=nl,
                 y_has_next=TRUE)
            step(2 * kk + 1, B, A, not_first=TRUE, not_last2=nl,
                 y_has_next=nl)

        drain_scatter(rowsB, dB, semSB)

        plsc.subcore_barrier()
        _copy_out(agg_sh, agg_hbm, s, t, NPAD)

    return k(h_flat, esrc, edst, ecode, combo_rep)


def _tc_linear(agg, W, b):
    def body(a0_ref, a1_ref, w_ref, b_ref, o_ref):
        x0 = a0_ref[...]
        x1 = a1_ref[...]
        y = jnp.dot(x0, w_ref[0:HALF, :], preferred_element_type=jnp.float32)
        y = y + jnp.dot(x1, w_ref[HALF:EMB, :],
                        preferred_element_type=jnp.float32)
        y = jnp.maximum(y + b_ref[...], 0.0)
        o_ref[0, :, :] = y[:, 0:HALF]
        o_ref[1, :, :] = y[:, HALF:EMB]

    return pl.pallas_call(
        body,
        grid=(25,),
        in_specs=[
            pl.BlockSpec((400, HALF), lambda i: (i, 0)),
            pl.BlockSpec((400, HALF), lambda i: (i + NPAD // 400, 0)),
            pl.BlockSpec((EMB, EMB), lambda i: (0, 0)),
            pl.BlockSpec((1, EMB), lambda i: (0, 0)),
        ],
        out_specs=pl.BlockSpec((2, 400, HALF), lambda i: (0, i, 0)),
        out_shape=jax.ShapeDtypeStruct((2, N_NODES, HALF), jnp.float32),
    )(agg, agg, W, b.reshape(1, EMB))


def _tc_last(agg, W, b, clf_W):
    def body(a0_ref, a1_ref, w_ref, b_ref, cw_ref, o_ref):
        i = pl.program_id(0)
        x0 = a0_ref[...]
        x1 = a1_ref[...]
        y = jnp.dot(x0, w_ref[0:HALF, :], preferred_element_type=jnp.float32)
        y = y + jnp.dot(x1, w_ref[HALF:EMB, :],
                        preferred_element_type=jnp.float32)
        y = jnp.maximum(y + b_ref[...], 0.0)
        p = y * cw_ref[...]
        acc = p[0:8, :]
        for r in range(1, 50):
            acc = acc + p[8 * r:8 * (r + 1), :]

        @pl.when(i == 0)
        def _():
            o_ref[...] = jnp.zeros((8, EMB), jnp.float32)

        o_ref[...] += acc

    return pl.pallas_call(
        body,
        grid=(25,),
        in_specs=[
            pl.BlockSpec((400, HALF), lambda i: (i, 0)),
            pl.BlockSpec((400, HALF), lambda i: (i + NPAD // 400, 0)),
            pl.BlockSpec((EMB, EMB), lambda i: (0, 0)),
            pl.BlockSpec((1, EMB), lambda i: (0, 0)),
            pl.BlockSpec((1, EMB), lambda i: (0, 0)),
        ],
        out_specs=pl.BlockSpec((8, EMB), lambda i: (0, 0)),
        out_shape=jax.ShapeDtypeStruct((8, EMB), jnp.float32),
    )(agg, agg, W, b.reshape(1, EMB), clf_W.reshape(1, EMB))


def kernel(node_feat, edge_feat, edge_index, atom_tables, bond_tables,
           lin_W, lin_b, clf_W, clf_b):
    i32 = jnp.int32

    asrc = (node_feat.astype(i32)
            + 100 * jnp.arange(9, dtype=i32)[None, :]).reshape(-1)
    adst = jnp.repeat(jnp.arange(N_NODES, dtype=i32), 9,
                      total_repeat_length=AE)
    apad = jnp.arange(AEP - AE, dtype=i32)
    asrc = jnp.concatenate([asrc, 900 + apad % 60])
    adst = jnp.concatenate([adst, N_NODES + apad % 16])

    atab = atom_tables.reshape(900, 2, HALF).transpose(1, 0, 2)
    atabf = jnp.concatenate(
        [atab, jnp.zeros((2, ATAB_ROWS - 900, HALF), jnp.float32)],
        axis=1).reshape(2 * ATAB_ROWS, HALF)

    ef = edge_feat.astype(i32)
    ecode = ef[:, 0] * 25 + ef[:, 1] * 5 + ef[:, 2]
    epad = jnp.arange(EP - N_EDGES, dtype=i32)
    esrc = jnp.concatenate([edge_index[0].astype(i32), epad % N_NODES])
    edst = jnp.concatenate([edge_index[1].astype(i32), N_NODES + epad % 16])
    ecode = jnp.concatenate([ecode, jnp.full((EP - N_EDGES,), 125, i32)])

    cb = (bond_tables[:, 0][:, :, None, None, :]
          + bond_tables[:, 1][:, None, :, None, :]
          + bond_tables[:, 2][:, None, None, :, :]).reshape(3, 125, EMB)
    cb = jnp.concatenate(
        [cb, jnp.zeros((3, COMBO_ROWS - 125, EMB), jnp.float32)], axis=1)
    cb = cb.reshape(3, COMBO_ROWS, 2, HALF).transpose(0, 2, 1, 3)
    cb = jnp.broadcast_to(cb[:, :, None], (3, 2, NS, COMBO_ROWS, HALF))
    combo_rep = cb.reshape(3, 2 * NS * COMBO_ROWS, HALF)

    h_flat = _atom_encode(atabf, asrc, adst)
    out8 = None
    for l in range(N_LAYERS):
        agg = _edge_layer(h_flat, esrc, edst, ecode, combo_rep[l])
        if l < N_LAYERS - 1:
            h3d = _tc_linear(agg, lin_W[l], lin_b[l])
            h_flat = h3d.reshape(2 * N_NODES, HALF)
        else:
            out8 = _tc_last(agg, lin_W[l], lin_b[l], clf_W)

    return jnp.sum(out8).reshape(1) + clf_b
